# baseline (speedup 1.0000x reference)
import functools

import jax
import jax.numpy as jnp
from jax import lax
from jax.experimental import pallas as pl
from jax.experimental.pallas import tpu as pltpu
from jax.experimental.pallas import tpu_sc as plsc

NUM_EMBEDDINGS = 1024
EMBEDDING_DIM = 64
COMMITMENT_COST = 0.25
TILE = 2048

NC, NS = 2, 16
NW = NC * NS
CHT = 432


def _pass_a(x_ref, w_ref, x2_ref, w2_ref, idx_ref, loss_ref, loss_acc,
            *, n_tok, n_steps):
    i = pl.program_id(0)

    @pl.when(i == 0)
    def _():
        loss_acc[0, 0] = 0.0

    mm = lax.dot_general(x_ref[...], w_ref[...], (((1,), (1,)), ((), ())),
                         preferred_element_type=jnp.float32)
    d = (x2_ref[...] + w2_ref[...]) - 2.0 * mm
    dmin = jnp.min(d, axis=1, keepdims=True)
    iota = lax.broadcasted_iota(jnp.int32, (TILE, NUM_EMBEDDINGS), 1)
    idx = jnp.min(jnp.where(d == dmin, iota, NUM_EMBEDDINGS), axis=1)
    idx_ref[...] = idx[:, None]
    loss_acc[0, 0] += jnp.sum(dmin)

    @pl.when(i == n_steps - 1)
    def _():
        loss_ref[...] = jnp.full(
            (1, 1),
            (1.0 + COMMITMENT_COST) * loss_acc[0, 0]
            / (n_tok * EMBEDDING_DIM))


def _pass_b(idx_ref, enc_ref, ppl_ref, cnt_acc, *, n_tok, n_steps):
    i = pl.program_id(0)

    @pl.when(i == 0)
    def _():
        cnt_acc[...] = jnp.zeros_like(cnt_acc)

    iota = lax.broadcasted_iota(jnp.int32, (TILE, NUM_EMBEDDINGS), 1)
    onehot = (iota == idx_ref[...]).astype(jnp.float32)
    enc_ref[...] = onehot
    cnt_acc[...] += jnp.sum(onehot, axis=0, keepdims=True)

    @pl.when(i == n_steps - 1)
    def _():
        avg = cnt_acc[...] * (1.0 / n_tok)
        ppl_ref[...] = jnp.exp(
            -jnp.sum(avg * jnp.log(avg + 1e-10), keepdims=True))


def _make_sq(n_tok):
    b_w = n_tok // NW
    n_ch = b_w // CHT
    mesh = plsc.VectorSubcoreMesh(core_axis_name="c", subcore_axis_name="s")

    @functools.partial(
        pl.kernel, mesh=mesh,
        out_type=jax.ShapeDtypeStruct((n_tok, 128), jnp.float32),
        scratch_types=[pltpu.VMEM((b_w,), jnp.int32),
                       pltpu.VMEM((CHT, 128), jnp.float32),
                       pltpu.VMEM((CHT, 128), jnp.float32),
                       pltpu.SemaphoreType.DMA,
                       pltpu.SemaphoreType.DMA])
    def sq(idx3_hbm, wpad_hbm, qpad_hbm, idx_v, buf0, buf1, sem0, sem1):
        wid = lax.axis_index("s") * NC + lax.axis_index("c")
        base = wid * b_w
        pltpu.sync_copy(idx3_hbm.at[wid], idx_v)
        bufs = (buf0, buf1)
        sems = (sem0, sem1)
        g = [None] * n_ch
        for c in range(n_ch):
            b = c % 2
            if c >= 2:
                g[c - 2].wait()
                pltpu.sync_copy(bufs[b], qpad_hbm.at[
                    pl.ds(base + (c - 2) * CHT, CHT)])
            g[c] = pltpu.async_copy(
                wpad_hbm.at[idx_v.at[pl.ds(c * CHT, CHT)]], bufs[b], sems[b])
        for c in (n_ch - 2, n_ch - 1):
            g[c].wait()
            pltpu.sync_copy(bufs[c % 2], qpad_hbm.at[
                pl.ds(base + c * CHT, CHT)])

    return sq


@jax.jit
def kernel(inputs, weight):
    bs, seq_len, feat = inputs.shape
    flat = inputs.reshape(-1, EMBEDDING_DIM)
    n_tok = flat.shape[0]
    n_steps = n_tok // TILE
    x2 = jnp.sum(flat ** 2, axis=1, keepdims=True)
    w2 = jnp.sum(weight ** 2, axis=1)[None, :]

    idx, loss = pl.pallas_call(
        functools.partial(_pass_a, n_tok=n_tok, n_steps=n_steps),
        grid=(n_steps,),
        in_specs=[
            pl.BlockSpec((TILE, EMBEDDING_DIM), lambda i: (i, 0)),
            pl.BlockSpec((NUM_EMBEDDINGS, EMBEDDING_DIM), lambda i: (0, 0)),
            pl.BlockSpec((TILE, 1), lambda i: (i, 0)),
            pl.BlockSpec((1, NUM_EMBEDDINGS), lambda i: (0, 0)),
        ],
        out_specs=[
            pl.BlockSpec((TILE, 1), lambda i: (i, 0)),
            pl.BlockSpec((1, 1), lambda i: (0, 0)),
        ],
        out_shape=[
            jax.ShapeDtypeStruct((n_tok, 1), jnp.int32),
            jax.ShapeDtypeStruct((1, 1), jnp.float32),
        ],
        scratch_shapes=[pltpu.SMEM((1, 1), jnp.float32)],
    )(flat, weight, x2, w2)

    wpad = jnp.pad(weight, ((0, 0), (0, 128 - EMBEDDING_DIM)))
    qpad = _make_sq(n_tok)(idx[:, 0].reshape(NW, n_tok // NW), wpad)

    enc, ppl = pl.pallas_call(
        functools.partial(_pass_b, n_tok=n_tok, n_steps=n_steps),
        grid=(n_steps,),
        in_specs=[pl.BlockSpec((TILE, 1), lambda i: (i, 0))],
        out_specs=[
            pl.BlockSpec((TILE, NUM_EMBEDDINGS), lambda i: (i, 0)),
            pl.BlockSpec((1, 1), lambda i: (0, 0)),
        ],
        out_shape=[
            jax.ShapeDtypeStruct((n_tok, NUM_EMBEDDINGS), jnp.float32),
            jax.ShapeDtypeStruct((1, 1), jnp.float32),
        ],
        scratch_shapes=[pltpu.VMEM((1, NUM_EMBEDDINGS), jnp.float32)],
    )(idx)

    q = qpad[:, :EMBEDDING_DIM]
    return (loss[0, 0], q.reshape(bs, seq_len, feat), ppl[0, 0], enc, idx)


# speedup vs baseline: 1.2070x; 1.2070x over previous
import functools

import jax
import jax.numpy as jnp
from jax.experimental import pallas as pl
from jax.experimental.pallas import tpu as pltpu

NUM_EMBEDDINGS = 1024
EMBEDDING_DIM = 64
COMMITMENT_COST = 0.25
TILE = 3456


def _vq_kernel(x_ref, w_ref, x2_ref, w2_ref, q_ref, enc_ref, idx_ref,
               loss_ref, ppl_ref, loss_acc, cnt_acc, *, n_tok, n_steps):
    i = pl.program_id(0)

    @pl.when(i == 0)
    def _init():
        loss_acc[0, 0] = 0.0
        cnt_acc[...] = jnp.zeros_like(cnt_acc)

    x = x_ref[...]
    w = w_ref[...]
    mm = jax.lax.dot_general(
        x, w, dimension_numbers=(((1,), (1,)), ((), ())),
        preferred_element_type=jnp.float32)
    d = (x2_ref[...] + w2_ref[...]) - 2.0 * mm
    dmin = jnp.min(d, axis=1, keepdims=True)
    iota = jax.lax.broadcasted_iota(jnp.int32, (TILE, NUM_EMBEDDINGS), 1)
    idx = jnp.min(jnp.where(d == dmin, iota, NUM_EMBEDDINGS), axis=1)
    onehot = (iota == idx[:, None]).astype(jnp.float32)
    q = jax.lax.dot_general(
        onehot, w, dimension_numbers=(((1,), (0,)), ((), ())),
        preferred_element_type=jnp.float32)

    enc_ref[...] = onehot
    q_ref[...] = q
    idx_ref[...] = idx[:, None]

    loss_acc[0, 0] += jnp.sum(dmin)
    cnt_acc[...] += jnp.sum(onehot, axis=0, keepdims=True)

    @pl.when(i == n_steps - 1)
    def _fin():
        n_elems = n_tok * EMBEDDING_DIM
        loss_ref[...] = jnp.full(
            (1, 1), (1.0 + COMMITMENT_COST) * loss_acc[0, 0] / n_elems)
        avg = cnt_acc[...] * (1.0 / n_tok)
        ppl_ref[...] = jnp.exp(
            -jnp.sum(avg * jnp.log(avg + 1e-10), keepdims=True))


@jax.jit
def kernel(inputs, weight):
    bs, seq_len, feat = inputs.shape
    flat = inputs.reshape(-1, EMBEDDING_DIM)
    n_tok = flat.shape[0]
    n_steps = n_tok // TILE
    x2 = jnp.sum(flat ** 2, axis=1, keepdims=True)
    w2 = jnp.sum(weight ** 2, axis=1)[None, :]

    q, enc, idx, loss, ppl = pl.pallas_call(
        functools.partial(_vq_kernel, n_tok=n_tok, n_steps=n_steps),
        grid=(n_steps,),
        in_specs=[
            pl.BlockSpec((TILE, EMBEDDING_DIM), lambda i: (i, 0)),
            pl.BlockSpec((NUM_EMBEDDINGS, EMBEDDING_DIM), lambda i: (0, 0)),
            pl.BlockSpec((TILE, 1), lambda i: (i, 0)),
            pl.BlockSpec((1, NUM_EMBEDDINGS), lambda i: (0, 0)),
        ],
        out_specs=[
            pl.BlockSpec((TILE, EMBEDDING_DIM), lambda i: (i, 0)),
            pl.BlockSpec((TILE, NUM_EMBEDDINGS), lambda i: (i, 0)),
            pl.BlockSpec((TILE, 1), lambda i: (i, 0)),
            pl.BlockSpec((1, 1), lambda i: (0, 0)),
            pl.BlockSpec((1, 1), lambda i: (0, 0)),
        ],
        out_shape=[
            jax.ShapeDtypeStruct((n_tok, EMBEDDING_DIM), jnp.float32),
            jax.ShapeDtypeStruct((n_tok, NUM_EMBEDDINGS), jnp.float32),
            jax.ShapeDtypeStruct((n_tok, 1), jnp.int32),
            jax.ShapeDtypeStruct((1, 1), jnp.float32),
            jax.ShapeDtypeStruct((1, 1), jnp.float32),
        ],
        scratch_shapes=[
            pltpu.SMEM((1, 1), jnp.float32),
            pltpu.VMEM((1, NUM_EMBEDDINGS), jnp.float32),
        ],
    )(flat, weight, x2, w2)

    return (loss[0, 0], q.reshape(bs, seq_len, feat), ppl[0, 0], enc, idx)


# speedup vs baseline: 1.2629x; 1.0463x over previous
import functools

import jax
import jax.numpy as jnp
from jax.experimental import pallas as pl
from jax.experimental.pallas import tpu as pltpu

NUM_EMBEDDINGS = 1024
EMBEDDING_DIM = 64
COMMITMENT_COST = 0.25
TILE = 3456


def _vq_kernel(x_ref, w_ref, x2_ref, w2_ref, q_ref, enc_ref, idx_ref,
               loss_ref, ppl_ref, loss_acc, cnt_acc, *, n_tok, n_steps):
    i = pl.program_id(0)

    @pl.when(i == 0)
    def _init():
        loss_acc[0, 0] = 0.0
        cnt_acc[...] = jnp.zeros_like(cnt_acc)

    x = x_ref[...]
    w = w_ref[...]
    mm = jax.lax.dot_general(
        x, w, dimension_numbers=(((1,), (1,)), ((), ())),
        preferred_element_type=jnp.float32)
    d = (x2_ref[...] + w2_ref[...]) - 2.0 * mm
    dmin = jnp.min(d, axis=1, keepdims=True)
    iota = jax.lax.broadcasted_iota(jnp.int32, (TILE, NUM_EMBEDDINGS), 1)
    idx = jnp.min(jnp.where(d == dmin, iota, NUM_EMBEDDINGS), axis=1)
    onehot = (iota == idx[:, None]).astype(jnp.float32)
    q = jax.lax.dot_general(
        onehot, w, dimension_numbers=(((1,), (0,)), ((), ())),
        preferred_element_type=jnp.float32)

    enc_ref[...] = onehot
    q_ref[...] = q
    idx_ref[...] = idx[:, None]

    loss_acc[0, 0] += jnp.sum(dmin)
    ones8 = jnp.ones((8, TILE), jnp.float32)
    cnt8 = jax.lax.dot_general(
        ones8, onehot, dimension_numbers=(((1,), (0,)), ((), ())),
        preferred_element_type=jnp.float32)
    cnt_acc[...] += jnp.sum(cnt8, axis=0, keepdims=True) * 0.125

    @pl.when(i == n_steps - 1)
    def _fin():
        n_elems = n_tok * EMBEDDING_DIM
        loss_ref[...] = jnp.full(
            (1, 1), (1.0 + COMMITMENT_COST) * loss_acc[0, 0] / n_elems)
        avg = cnt_acc[...] * (1.0 / n_tok)
        ppl_ref[...] = jnp.exp(
            -jnp.sum(avg * jnp.log(avg + 1e-10), keepdims=True))


@jax.jit
def kernel(inputs, weight):
    bs, seq_len, feat = inputs.shape
    flat = inputs.reshape(-1, EMBEDDING_DIM)
    n_tok = flat.shape[0]
    n_steps = n_tok // TILE
    x2 = jnp.sum(flat ** 2, axis=1, keepdims=True)
    w2 = jnp.sum(weight ** 2, axis=1)[None, :]

    q, enc, idx, loss, ppl = pl.pallas_call(
        functools.partial(_vq_kernel, n_tok=n_tok, n_steps=n_steps),
        grid=(n_steps,),
        in_specs=[
            pl.BlockSpec((TILE, EMBEDDING_DIM), lambda i: (i, 0)),
            pl.BlockSpec((NUM_EMBEDDINGS, EMBEDDING_DIM), lambda i: (0, 0)),
            pl.BlockSpec((TILE, 1), lambda i: (i, 0)),
            pl.BlockSpec((1, NUM_EMBEDDINGS), lambda i: (0, 0)),
        ],
        out_specs=[
            pl.BlockSpec((TILE, EMBEDDING_DIM), lambda i: (i, 0)),
            pl.BlockSpec((TILE, NUM_EMBEDDINGS), lambda i: (i, 0)),
            pl.BlockSpec((TILE, 1), lambda i: (i, 0)),
            pl.BlockSpec((1, 1), lambda i: (0, 0)),
            pl.BlockSpec((1, 1), lambda i: (0, 0)),
        ],
        out_shape=[
            jax.ShapeDtypeStruct((n_tok, EMBEDDING_DIM), jnp.float32),
            jax.ShapeDtypeStruct((n_tok, NUM_EMBEDDINGS), jnp.float32),
            jax.ShapeDtypeStruct((n_tok, 1), jnp.int32),
            jax.ShapeDtypeStruct((1, 1), jnp.float32),
            jax.ShapeDtypeStruct((1, 1), jnp.float32),
        ],
        scratch_shapes=[
            pltpu.SMEM((1, 1), jnp.float32),
            pltpu.VMEM((1, NUM_EMBEDDINGS), jnp.float32),
        ],
    )(flat, weight, x2, w2)

    return (loss[0, 0], q.reshape(bs, seq_len, feat), ppl[0, 0], enc, idx)
